# baseline (speedup 1.0000x reference)
import functools

import jax
import jax.numpy as jnp
from jax import lax
from jax.experimental import pallas as pl
from jax.experimental.pallas import tpu as pltpu
from jax.experimental.pallas import tpu_sc as plsc

N = 10000
E = 320000
D = 128
DE = 144
G = 64
C = 2
N_GRAPH_LAYER = 2
N_FC_LAYER = 2

E2 = E + N
NW = 32
SB = 32
NGRP = 81
NB = 4 * NGRP
EPW = NB * SB
E_PAD = NW * EPW

_MESH = plsc.VectorSubcoreMesh(
    core_axis_name="c", subcore_axis_name="s", num_cores=2, num_subcores=16)

RSTRIPE = N // 16


def _edge_body(as_hbm, ad_hbm, sd_hbm, h_hbm,
               acc_out,
               as_v, ad_v, ib, exs, rows,
               acc, gsem, ssem):
    cid = lax.axis_index("c")
    sid = lax.axis_index("s")
    wid = sid * 2 + cid

    pltpu.sync_copy(as_hbm, as_v)
    pltpu.sync_copy(ad_hbm, ad_v)

    zeros16 = jnp.zeros((16,), jnp.float32)

    def _zb(i, _):
        rows[0, i // 9, pl.ds((i % 9) * 16, 16)] = zeros16
        return 0
    lax.fori_loop(0, SB * 9, _zb, 0)

    for k in range(RSTRIPE // 25):
        pltpu.sync_copy(rows.at[0].at[pl.ds(0, 25)],
                        acc.at[pl.ds(sid * RSTRIPE + k * 25, 25)])
    plsc.subcore_barrier()

    ebase = wid * EPW
    iota16 = lax.broadcasted_iota(jnp.int32, (16,), 0)

    def _idx(j):
        return ib.at[lax.rem(lax.div(j, 4), 3), lax.rem(j, 4)]

    def _gather(j):
        b = lax.rem(j, 4)
        pltpu.async_copy(h_hbm.at[_idx(j).at[0]], rows.at[b], gsem.at[b])

    def _wait_gather(j):
        b = lax.rem(j, 4)
        pltpu.make_async_copy(h_hbm.at[_idx(j).at[0]], rows.at[b],
                              gsem.at[b]).wait()

    def _wait_scatter(b):
        pltpu.make_async_copy(rows.at[b], acc.at[_idx(0).at[1]],
                              ssem.at[b]).wait()

    pltpu.sync_copy(sd_hbm.at[wid, 0], ib.at[0])
    _gather(0)
    _gather(1)

    def _batch(j, _):
        b = lax.rem(j, 4)
        ibj = _idx(j)
        @pl.when((lax.rem(j, 4) == 0) & (j + 4 < NB))
        def _():
            pltpu.sync_copy(sd_hbm.at[wid, lax.div(j, 4) + 1],
                            ib.at[lax.rem(lax.div(j, 4) + 1, 3)])

        for g in range(SB // 16):
            sl = pl.ds(g * 16, 16)
            es = plsc.load_gather(as_v, [ibj[0, sl]])
            ed = plsc.load_gather(ad_v, [ibj[1, sl]])
            e = es + ed
            e = jnp.maximum(e, 0.2 * e)
            ex = jnp.exp(e)
            gidx = ebase + j * SB + g * 16 + iota16
            exs[b, sl] = jnp.where(gidx < E2, ex, 0.0)

        _wait_gather(j)

        @pl.when(j + 2 < NB)
        def _():
            @pl.when(j >= 2)
            def _():
                _wait_scatter(lax.rem(j + 2, 4))
            _gather(j + 2)

        rj = rows.at[b]

        for g in range(SB // 16):
            exg = exs[b, pl.ds(g * 16, 16)]
            for l in range(16):
                i = g * 16 + l
                exv = exg[l]
                for kk in range(9):
                    sl = pl.ds(kk * 16, 16)
                    rj[i, sl] = rj[i, sl] * exv

        pltpu.async_copy(rj, acc.at[ibj.at[1]], ssem.at[b], add=True)
        return 0

    lax.fori_loop(0, NB, _batch, 0)
    for b in range(4):
        _wait_scatter(b)
    plsc.subcore_barrier()

    pltpu.sync_copy(acc.at[pl.ds(sid * RSTRIPE, RSTRIPE)],
                    acc_out.at[cid, sid])


_edge_kernel = functools.partial(
    pl.kernel,
    out_type=jax.ShapeDtypeStruct((2, 16, RSTRIPE, DE), jnp.float32),
    mesh=_MESH,
    compiler_params=pltpu.CompilerParams(needs_layout_passes=False,
                                         use_tc_tiling_on_sc=False),
    scratch_types=[
        pltpu.VMEM((N,), jnp.float32),
        pltpu.VMEM((N,), jnp.float32),
        pltpu.VMEM((3, 4, 2, SB), jnp.int32),
        pltpu.VMEM((4, SB), jnp.float32),
        pltpu.VMEM((4, SB, DE), jnp.float32),
        pltpu.VMEM_SHARED((N, DE), jnp.float32),
        pltpu.SemaphoreType.DMA((4,)),
        pltpu.SemaphoreType.DMA((4,)),
    ],
)(_edge_body)


def _ext(h):
    one = jnp.ones((N, 1), jnp.float32)
    zero = jnp.zeros((N, DE - D - 1), jnp.float32)
    return jnp.concatenate([h, one, zero], axis=1)


def _mm_first_body(x_ref, W_ref, as_ref, ad_ref, h_ref, asv_ref, adv_ref):
    h = jnp.dot(x_ref[...], W_ref[...], preferred_element_type=jnp.float32)
    h_ref[...] = _ext(h)
    asv_ref[...] = jnp.dot(h, as_ref[...], preferred_element_type=jnp.float32)
    adv_ref[...] = jnp.dot(h, ad_ref[...], preferred_element_type=jnp.float32)


def _mm_combine_body(accp_ref, b_ref, W_ref, as_ref, ad_ref,
                     h_ref, asv_ref, adv_ref):
    num = accp_ref[0, :, :D] + accp_ref[1, :, :D]
    den = accp_ref[0, :, D] + accp_ref[1, :, D]
    hprev = jax.nn.relu(num / (den[:, None] + 1e-16) + b_ref[...])
    h = jnp.dot(hprev, W_ref[...], preferred_element_type=jnp.float32)
    h_ref[...] = _ext(h)
    asv_ref[...] = jnp.dot(h, as_ref[...], preferred_element_type=jnp.float32)
    adv_ref[...] = jnp.dot(h, ad_ref[...], preferred_element_type=jnp.float32)


def _head_body(accp_ref, b_ref, batchs_ref,
               Wl1_ref, bl1_ref, Wls_ref, bls_ref, Wl3_ref, bl3_ref, out_ref):
    num = accp_ref[0, :, :D] + accp_ref[1, :, :D]
    den = accp_ref[0, :, D] + accp_ref[1, :, D]
    h = jax.nn.relu(num / (den[:, None] + 1e-16) + b_ref[...])
    batchs = batchs_ref[...]
    gids = lax.broadcasted_iota(jnp.int32, (G, N), 0)
    onehot = (gids == batchs[None, :]).astype(jnp.float32)
    sums = jnp.dot(onehot, h, preferred_element_type=jnp.float32)
    cnt = jnp.sum(onehot, axis=1)
    p = sums / jnp.clip(cnt, 1.0)[:, None]
    p = jax.nn.relu(jnp.dot(p, Wl1_ref[...],
                            preferred_element_type=jnp.float32) + bl1_ref[...])
    for i in range(N_FC_LAYER):
        p = jax.nn.relu(jnp.dot(p, Wls_ref[i],
                                preferred_element_type=jnp.float32) + bls_ref[i])
    p = jnp.dot(p, Wl3_ref[...], preferred_element_type=jnp.float32) + bl3_ref[...]
    m = jnp.max(p, axis=1, keepdims=True)
    lse = jnp.log(jnp.sum(jnp.exp(p - m), axis=1, keepdims=True)) + m
    out_ref[...] = p - lse


def _mm_first(x, W, a_s, a_d):
    return pl.pallas_call(
        _mm_first_body,
        out_shape=[
            jax.ShapeDtypeStruct((N, DE), jnp.float32),
            jax.ShapeDtypeStruct((N, 1), jnp.float32),
            jax.ShapeDtypeStruct((N, 1), jnp.float32),
        ],
    )(x, W, a_s.reshape(D, 1), a_d.reshape(D, 1))


def _mm_combine(accp, b, W, a_s, a_d):
    return pl.pallas_call(
        _mm_combine_body,
        out_shape=[
            jax.ShapeDtypeStruct((N, DE), jnp.float32),
            jax.ShapeDtypeStruct((N, 1), jnp.float32),
            jax.ShapeDtypeStruct((N, 1), jnp.float32),
        ],
    )(accp.reshape(2, N, DE), b, W, a_s.reshape(D, 1), a_d.reshape(D, 1))


def _head(accp, b, batchs, Wl1, bl1, Wls, bls, Wl3, bl3):
    return pl.pallas_call(
        _head_body,
        out_shape=jax.ShapeDtypeStruct((G, C), jnp.float32),
    )(accp.reshape(2, N, DE), b, batchs, Wl1, bl1, Wls, bls, Wl3, bl3)


def kernel(x, edge_index, batchs, W1, as1, ad1, b1, Wg, asg, adg, bg,
           Wl1, bl1, Wls, bls, Wl3, bl3):
    loop = jnp.arange(N, dtype=edge_index.dtype)
    pad = jnp.zeros((E_PAD - E2,), edge_index.dtype)
    src = jnp.concatenate([edge_index[0], loop, pad]).reshape(NW, NGRP, 4, 1, SB)
    dst = jnp.concatenate([edge_index[1], loop, pad]).reshape(NW, NGRP, 4, 1, SB)
    sd = jnp.concatenate([src, dst], axis=3)

    h, asv, adv = _mm_first(x, W1, as1, ad1)
    accp = _edge_kernel(asv.reshape(N), adv.reshape(N), sd, h)
    for i in range(N_GRAPH_LAYER):
        h, asv, adv = _mm_combine(accp, b1 if i == 0 else bg[i - 1],
                                  Wg[i], asg[i], adg[i])
        accp = _edge_kernel(asv.reshape(N), adv.reshape(N), sd, h)
    return _head(accp, bg[N_GRAPH_LAYER - 1], batchs,
                 Wl1, bl1, Wls, bls, Wl3, bl3)
